# baseline (speedup 1.0000x reference)
import functools

import jax
import jax.numpy as jnp
from jax import lax
from jax.experimental import pallas as pl
from jax.experimental.pallas import tpu as pltpu
from jax.experimental.pallas import tpu_sc as plsc

N = 7
NP = 8
E = 32
F0 = 224
H1 = 64
H2 = 256
OUT = 576


def _sc_adj(ei_hbm, out_hbm, ei_v, dinv_v, deg_v, a_v):
    f32 = jnp.float32
    core = lax.axis_index("c")
    sub = lax.axis_index("s")

    @pl.when((core == 0) & (sub == 0))
    def _():
        pltpu.sync_copy(ei_hbm, ei_v)
        deg_v[...] = jnp.zeros((16,), f32)
        for i in range(4):
            a_v[pl.ds(i * 16, 16)] = jnp.zeros((16,), f32)
        row0 = ei_v[pl.ds(0, 16)]
        row1 = ei_v[pl.ds(16, 16)]
        col0 = ei_v[pl.ds(32, 16)]
        col1 = ei_v[pl.ds(48, 16)]
        ones = jnp.ones((16,), f32)
        plsc.addupdate_scatter(deg_v, [col0], ones)
        plsc.addupdate_scatter(deg_v, [col1], ones)
        nodes = lax.broadcasted_iota(jnp.int32, (16,), 0)
        deg = deg_v[...] + jnp.where(nodes < N, 1.0, 0.0)
        t = jnp.maximum(deg, 1.0)
        y = 1.0 / t
        for _ in range(10):
            y = y * (1.5 - 0.5 * t * y * y)
        dinv = jnp.where(deg > 0, y, 0.0)
        dinv_v[...] = dinv
        norm0 = plsc.load_gather(dinv_v, [row0]) * plsc.load_gather(dinv_v, [col0])
        norm1 = plsc.load_gather(dinv_v, [row1]) * plsc.load_gather(dinv_v, [col1])
        plsc.addupdate_scatter(a_v, [col0 * NP + row0], norm0)
        plsc.addupdate_scatter(a_v, [col1 * NP + row1], norm1)
        plsc.addupdate_scatter(a_v, [nodes * (NP + 1)], dinv * dinv,
                               mask=nodes < N)
        pltpu.sync_copy(a_v, out_hbm)


def _sc_adjacency(edge_index):
    mesh = plsc.VectorSubcoreMesh(core_axis_name="c", subcore_axis_name="s")
    k = functools.partial(
        pl.kernel,
        mesh=mesh,
        compiler_params=pltpu.CompilerParams(needs_layout_passes=False),
        out_type=jax.ShapeDtypeStruct((NP * NP,), jnp.float32),
        scratch_types=[
            pltpu.VMEM((2 * E,), jnp.int32),
            pltpu.VMEM((16,), jnp.float32),
            pltpu.VMEM((16,), jnp.float32),
            pltpu.VMEM((NP * NP,), jnp.float32),
        ],
    )(_sc_adj)
    return k(edge_index.reshape(2 * E))


def _gnn_tc(a_hbm, x_hbm, w1_hbm, b1_hbm, w2_hbm, b2_hbm,
            wfc_hbm, bfc_hbm, out_ref,
            a_s, x_s, w1_s, b1_s, w2_s, b2_s, wfc_s, bfc_s, sems):
    f32 = jnp.float32
    cps = [
        pltpu.make_async_copy(wfc_hbm, wfc_s, sems.at[0]),
        pltpu.make_async_copy(a_hbm, a_s, sems.at[1]),
        pltpu.make_async_copy(x_hbm, x_s, sems.at[2]),
        pltpu.make_async_copy(w1_hbm, w1_s, sems.at[3]),
        pltpu.make_async_copy(b1_hbm, b1_s, sems.at[4]),
        pltpu.make_async_copy(w2_hbm, w2_s, sems.at[5]),
        pltpu.make_async_copy(b2_hbm, b2_s, sems.at[6]),
        pltpu.make_async_copy(bfc_hbm, bfc_s, sems.at[7]),
    ]
    for c in cps:
        c.start()
    for c in cps[1:]:
        c.wait()

    A = a_s[...]
    xw1 = jax.lax.dot_general(x_s[...], w1_s[...],
                              (((0,), (0,)), ((), ())),
                              preferred_element_type=f32)
    xw1 = jnp.pad(xw1, ((0, NP - N), (0, 0)))
    h1 = jax.nn.relu(jnp.dot(A, xw1, preferred_element_type=f32)
                     + b1_s[...])
    xw2 = jnp.dot(h1, w2_s[...], preferred_element_type=f32)
    h2 = jax.nn.relu(jnp.dot(A, xw2, preferred_element_type=f32)
                     + b2_s[...])

    cps[0].wait()
    acc = bfc_s[...]
    for n in range(N):
        acc = acc + jnp.dot(h2[n:n + 1, :],
                            wfc_s[pl.ds(n * H2, H2), :],
                            preferred_element_type=f32)
    out_ref[...] = acc


def kernel(x, edge_index, W1, b1, W2, b2, Wfc, bfc):
    A = _sc_adjacency(edge_index.astype(jnp.int32)).reshape(NP, NP)
    any_spec = pl.BlockSpec(memory_space=pl.ANY)
    out = pl.pallas_call(
        _gnn_tc,
        in_specs=[any_spec] * 8,
        out_specs=pl.BlockSpec((1, OUT), lambda: (0, 0)),
        out_shape=jax.ShapeDtypeStruct((1, OUT), jnp.float32),
        scratch_shapes=[
            pltpu.VMEM((NP, NP), jnp.float32),
            pltpu.VMEM((F0, N), jnp.float32),
            pltpu.VMEM((F0, H1), jnp.float32),
            pltpu.VMEM((1, H1), jnp.float32),
            pltpu.VMEM((H1, H2), jnp.float32),
            pltpu.VMEM((1, H2), jnp.float32),
            pltpu.VMEM((N * H2, OUT), jnp.float32),
            pltpu.VMEM((1, OUT), jnp.float32),
            pltpu.SemaphoreType.DMA((8,)),
        ],
    )(A, x, W1, b1.reshape(1, H1), W2, b2.reshape(1, H2),
      Wfc, bfc.reshape(1, OUT))
    return out.reshape(24, 24)


# speedup vs baseline: 2.0272x; 2.0272x over previous
import jax
import jax.numpy as jnp
from jax.experimental import pallas as pl
from jax.experimental.pallas import tpu as pltpu

N = 7
NP = 8
E = 32
F0 = 224
H1 = 64
H2 = 256
OUT = 576


def _gnn_kernel(ei_hbm, x_hbm, w1_hbm, b1_hbm, w2_hbm, b2_hbm,
                wfc_hbm, bfc_hbm, out_ref,
                ei_s, x_s, w1_s, b1_s, w2_s, b2_s, wfc_s, bfc_s, sems):
    f32 = jnp.float32
    wfc_cps = [
        pltpu.make_async_copy(wfc_hbm.at[pl.ds(n * H2, H2), :],
                              wfc_s.at[pl.ds(n * H2, H2), :],
                              sems.at[n])
        for n in range(N)
    ]
    cps = [
        pltpu.make_async_copy(ei_hbm, ei_s.at[pl.ds(0, 2), pl.ds(0, E)],
                              sems.at[N]),
        pltpu.make_async_copy(x_hbm, x_s, sems.at[N + 1]),
        pltpu.make_async_copy(w1_hbm, w1_s, sems.at[N + 2]),
        pltpu.make_async_copy(b1_hbm, b1_s, sems.at[N + 3]),
        pltpu.make_async_copy(w2_hbm, w2_s, sems.at[N + 4]),
        pltpu.make_async_copy(b2_hbm, b2_s, sems.at[N + 5]),
        pltpu.make_async_copy(bfc_hbm, bfc_s, sems.at[N + 6]),
    ]
    for c in wfc_cps:
        c.start()
    for c in cps:
        c.start()
    for c in cps:
        c.wait()

    row = ei_s[0:1, :E]
    col = ei_s[1:2, :E]
    nodes = jax.lax.broadcasted_iota(jnp.int32, (NP, 1), 0)
    ohr = (nodes == row).astype(f32)
    ohc = (nodes == col).astype(f32)
    real = (nodes < N).astype(f32)
    deg = jnp.sum(ohc, axis=1, keepdims=True) + real
    dinv = jnp.where(deg > 0, jax.lax.rsqrt(jnp.maximum(deg, 1e-12)), 0.0)
    dinv_row = jnp.sum(ohr * dinv, axis=0, keepdims=True)
    dinv_col = jnp.sum(ohc * dinv, axis=0, keepdims=True)
    norm = dinv_row * dinv_col
    A = jax.lax.dot_general(ohc * norm, ohr,
                            (((1,), (1,)), ((), ())),
                            preferred_element_type=f32)
    eye = (nodes == jax.lax.broadcasted_iota(jnp.int32, (1, NP), 1)
           ).astype(f32)
    A = A + eye * (dinv * dinv) * real

    xw1 = jax.lax.dot_general(x_s[...], w1_s[...],
                              (((0,), (0,)), ((), ())),
                              preferred_element_type=f32)
    xw1 = jnp.pad(xw1, ((0, NP - N), (0, 0)))
    h1 = jax.nn.relu(jnp.dot(A, xw1, preferred_element_type=f32)
                     + b1_s[...])
    xw2 = jnp.dot(h1, w2_s[...], preferred_element_type=f32)
    h2 = jax.nn.relu(jnp.dot(A, xw2, preferred_element_type=f32)
                     + b2_s[...])

    acc = bfc_s[...]
    for n in range(N):
        wfc_cps[n].wait()
        acc = acc + jnp.dot(h2[n:n + 1, :],
                            wfc_s[pl.ds(n * H2, H2), :],
                            preferred_element_type=f32)
    out_ref[...] = acc


def kernel(x, edge_index, W1, b1, W2, b2, Wfc, bfc):
    any_spec = pl.BlockSpec(memory_space=pl.ANY)
    out = pl.pallas_call(
        _gnn_kernel,
        in_specs=[any_spec] * 8,
        out_specs=pl.BlockSpec((1, OUT), lambda: (0, 0)),
        out_shape=jax.ShapeDtypeStruct((1, OUT), jnp.float32),
        scratch_shapes=[
            pltpu.VMEM((8, E), jnp.int32),
            pltpu.VMEM((F0, N), jnp.float32),
            pltpu.VMEM((F0, H1), jnp.float32),
            pltpu.VMEM((1, H1), jnp.float32),
            pltpu.VMEM((H1, H2), jnp.float32),
            pltpu.VMEM((1, H2), jnp.float32),
            pltpu.VMEM((N * H2, OUT), jnp.float32),
            pltpu.VMEM((1, OUT), jnp.float32),
            pltpu.SemaphoreType.DMA((N + 7,)),
        ],
    )(edge_index, x, W1, b1.reshape(1, H1), W2, b2.reshape(1, H2),
      Wfc, bfc.reshape(1, OUT))
    return out.reshape(24, 24)


# speedup vs baseline: 2.1015x; 1.0367x over previous
import jax
import jax.numpy as jnp
from jax.experimental import pallas as pl
from jax.experimental.pallas import tpu as pltpu

N = 7
NP = 8
E = 32
F0 = 224
H1 = 64
H2 = 256
OUT = 576


def _gnn_kernel(ei_hbm, x_hbm, w1_hbm, b1_hbm, w2_hbm, b2_hbm,
                wfc_hbm, bfc_hbm, out_ref,
                ei_s, x_s, w1_s, b1_s, w2_s, b2_s, wfc_s, bfc_s, sems):
    f32 = jnp.float32
    split = 3 * H2
    wfc_cps = [
        pltpu.make_async_copy(wfc_hbm.at[pl.ds(0, split), :],
                              wfc_s.at[pl.ds(0, split), :], sems.at[0]),
        pltpu.make_async_copy(wfc_hbm.at[pl.ds(split, N * H2 - split), :],
                              wfc_s.at[pl.ds(split, N * H2 - split), :],
                              sems.at[1]),
    ]
    cps = [
        pltpu.make_async_copy(ei_hbm, ei_s.at[pl.ds(0, 2), pl.ds(0, E)],
                              sems.at[2]),
        pltpu.make_async_copy(x_hbm, x_s, sems.at[3]),
        pltpu.make_async_copy(w1_hbm, w1_s, sems.at[4]),
        pltpu.make_async_copy(b1_hbm, b1_s, sems.at[5]),
        pltpu.make_async_copy(w2_hbm, w2_s, sems.at[6]),
        pltpu.make_async_copy(b2_hbm, b2_s, sems.at[7]),
        pltpu.make_async_copy(bfc_hbm, bfc_s, sems.at[8]),
    ]
    for c in wfc_cps:
        c.start()
    for c in cps:
        c.start()
    for c in cps:
        c.wait()

    row = ei_s[0:1, :E]
    col = ei_s[1:2, :E]
    nodes = jax.lax.broadcasted_iota(jnp.int32, (NP, 1), 0)
    ohr = (nodes == row).astype(f32)
    ohc = (nodes == col).astype(f32)
    real = (nodes < N).astype(f32)
    deg = jnp.sum(ohc, axis=1, keepdims=True) + real
    dinv = jnp.where(deg > 0, jax.lax.rsqrt(jnp.maximum(deg, 1e-12)), 0.0)
    dinv_row = jnp.sum(ohr * dinv, axis=0, keepdims=True)
    dinv_col = jnp.sum(ohc * dinv, axis=0, keepdims=True)
    norm = dinv_row * dinv_col
    A = jax.lax.dot_general(ohc * norm, ohr,
                            (((1,), (1,)), ((), ())),
                            preferred_element_type=f32)
    eye = (nodes == jax.lax.broadcasted_iota(jnp.int32, (1, NP), 1)
           ).astype(f32)
    A = A + eye * (dinv * dinv) * real

    xw1 = jax.lax.dot_general(x_s[...], w1_s[...],
                              (((0,), (0,)), ((), ())),
                              preferred_element_type=f32)
    xw1 = jnp.pad(xw1, ((0, NP - N), (0, 0)))
    h1 = jax.nn.relu(jnp.dot(A, xw1, preferred_element_type=f32)
                     + b1_s[...])
    xw2 = jnp.dot(h1, w2_s[...], preferred_element_type=f32)
    h2 = jax.nn.relu(jnp.dot(A, xw2, preferred_element_type=f32)
                     + b2_s[...])

    acc = bfc_s[...]
    for n in range(N):
        if n == 0:
            wfc_cps[0].wait()
        if n == 3:
            wfc_cps[1].wait()
        acc = acc + jnp.dot(h2[n:n + 1, :],
                            wfc_s[pl.ds(n * H2, H2), :],
                            preferred_element_type=f32)
    out_ref[...] = acc


def kernel(x, edge_index, W1, b1, W2, b2, Wfc, bfc):
    any_spec = pl.BlockSpec(memory_space=pl.ANY)
    out = pl.pallas_call(
        _gnn_kernel,
        in_specs=[any_spec] * 8,
        out_specs=pl.BlockSpec((1, OUT), lambda: (0, 0)),
        out_shape=jax.ShapeDtypeStruct((1, OUT), jnp.float32),
        scratch_shapes=[
            pltpu.VMEM((8, E), jnp.int32),
            pltpu.VMEM((F0, N), jnp.float32),
            pltpu.VMEM((F0, H1), jnp.float32),
            pltpu.VMEM((1, H1), jnp.float32),
            pltpu.VMEM((H1, H2), jnp.float32),
            pltpu.VMEM((1, H2), jnp.float32),
            pltpu.VMEM((N * H2, OUT), jnp.float32),
            pltpu.VMEM((1, OUT), jnp.float32),
            pltpu.SemaphoreType.DMA((9,)),
        ],
    )(edge_index, x, W1, b1.reshape(1, H1), W2, b2.reshape(1, H2),
      Wfc, bfc.reshape(1, OUT))
    return out.reshape(24, 24)


# speedup vs baseline: 2.1269x; 1.0121x over previous
import jax
import jax.numpy as jnp
from jax.experimental import pallas as pl
from jax.experimental.pallas import tpu as pltpu

N = 7
NP = 8
E = 32
F0 = 224
H1 = 64
H2 = 256
OUT = 576


def _gnn_kernel(ei_hbm, x_hbm, w1_hbm, b1_hbm, w2_hbm, b2_hbm,
                wfc_hbm, bfc_hbm, out_ref,
                ei_s, x_s, w1_s, b1_s, w2_s, b2_s, wfc_s, bfc_s, sems):
    f32 = jnp.float32
    cps = [
        pltpu.make_async_copy(wfc_hbm, wfc_s, sems.at[0]),
        pltpu.make_async_copy(ei_hbm, ei_s.at[pl.ds(0, 2), pl.ds(0, E)],
                              sems.at[1]),
        pltpu.make_async_copy(x_hbm, x_s, sems.at[2]),
        pltpu.make_async_copy(w1_hbm, w1_s, sems.at[3]),
        pltpu.make_async_copy(b1_hbm, b1_s, sems.at[4]),
        pltpu.make_async_copy(w2_hbm, w2_s, sems.at[5]),
        pltpu.make_async_copy(b2_hbm, b2_s, sems.at[6]),
        pltpu.make_async_copy(bfc_hbm, bfc_s, sems.at[7]),
    ]
    for c in cps:
        c.start()
    for c in cps[1:]:
        c.wait()

    row = ei_s[0:1, :E]
    col = ei_s[1:2, :E]
    nodes = jax.lax.broadcasted_iota(jnp.int32, (NP, 1), 0)
    ohr = (nodes == row).astype(f32)
    ohc = (nodes == col).astype(f32)
    real = (nodes < N).astype(f32)
    deg = jnp.sum(ohc, axis=1, keepdims=True) + real
    dinv = jnp.where(deg > 0, jax.lax.rsqrt(jnp.maximum(deg, 1e-12)), 0.0)
    dinv_row = jnp.sum(ohr * dinv, axis=0, keepdims=True)
    dinv_col = jnp.sum(ohc * dinv, axis=0, keepdims=True)
    norm = dinv_row * dinv_col
    A = jax.lax.dot_general(ohc * norm, ohr,
                            (((1,), (1,)), ((), ())),
                            preferred_element_type=f32)
    eye = (nodes == jax.lax.broadcasted_iota(jnp.int32, (1, NP), 1)
           ).astype(f32)
    A = A + eye * (dinv * dinv) * real

    xw1 = jax.lax.dot_general(x_s[...], w1_s[...],
                              (((0,), (0,)), ((), ())),
                              preferred_element_type=f32)
    xw1 = jnp.pad(xw1, ((0, NP - N), (0, 0)))
    h1 = jax.nn.relu(jnp.dot(A, xw1, preferred_element_type=f32)
                     + b1_s[...])
    xw2 = jnp.dot(h1, w2_s[...], preferred_element_type=f32)
    h2 = jax.nn.relu(jnp.dot(A, xw2, preferred_element_type=f32)
                     + b2_s[...])

    cps[0].wait()
    acc = bfc_s[...]
    for n in range(N):
        acc = acc + jnp.dot(h2[n:n + 1, :],
                            wfc_s[pl.ds(n * H2, H2), :],
                            preferred_element_type=f32)
    out_ref[...] = acc


def kernel(x, edge_index, W1, b1, W2, b2, Wfc, bfc):
    any_spec = pl.BlockSpec(memory_space=pl.ANY)
    out = pl.pallas_call(
        _gnn_kernel,
        in_specs=[any_spec] * 8,
        out_specs=pl.BlockSpec((1, OUT), lambda: (0, 0)),
        out_shape=jax.ShapeDtypeStruct((1, OUT), jnp.float32),
        scratch_shapes=[
            pltpu.VMEM((8, E), jnp.int32),
            pltpu.VMEM((F0, N), jnp.float32),
            pltpu.VMEM((F0, H1), jnp.float32),
            pltpu.VMEM((1, H1), jnp.float32),
            pltpu.VMEM((H1, H2), jnp.float32),
            pltpu.VMEM((1, H2), jnp.float32),
            pltpu.VMEM((N * H2, OUT), jnp.float32),
            pltpu.VMEM((1, OUT), jnp.float32),
            pltpu.SemaphoreType.DMA((8,)),
        ],
    )(edge_index, x, W1, b1.reshape(1, H1), W2, b2.reshape(1, H2),
      Wfc, bfc.reshape(1, OUT))
    return out.reshape(24, 24)
